# baseline (speedup 1.0000x reference)
import functools

import jax
import jax.numpy as jnp
from jax import lax
from jax.experimental import pallas as pl
from jax.experimental.pallas import tpu as pltpu
from jax.experimental.pallas import tpu_sc as plsc


def _make_gather(S0: int, S1: int, V: int, D: int, spc: int):
    info = plsc.get_sparse_core_info()
    NC, NS = info.num_cores, info.num_subcores
    NW = NC * NS
    B = S0 * S1
    chunk = spc * S1
    b_per_w = B // NW
    n_chunks = b_per_w // chunk
    assert B % (NW * chunk) == 0 and n_chunks % 2 == 0
    assert S1 % 8 == 0 and b_per_w % 8 == 0
    pieces = []
    off = 0
    while off < S1:
        w = min(128, S1 - off)
        pieces.append((off, w))
        off += w

    mesh = plsc.VectorSubcoreMesh(core_axis_name="c", subcore_axis_name="s")

    @functools.partial(
        pl.kernel,
        mesh=mesh,
        compiler_params=pltpu.CompilerParams(use_tc_tiling_on_sc=False),
        out_type=jax.ShapeDtypeStruct((S0, S1, D), jnp.float32),
        scratch_types=[
            pltpu.VMEM((b_per_w,), jnp.int32),
            pltpu.VMEM((spc, S1, D), jnp.float32),
            pltpu.VMEM((spc, S1, D), jnp.float32),
            pltpu.SemaphoreType.DMA,
            pltpu.SemaphoreType.DMA,
            pltpu.SemaphoreType.DMA,
            pltpu.SemaphoreType.DMA,
        ],
    )
    def gather(table_hbm, idx_hbm, out_hbm, idx_v, rows0, rows1,
               gsem0, gsem1, ssem0, ssem1):
        rows = (rows0, rows1)
        gsem = (gsem0, gsem1)
        ssem = (ssem0, ssem1)

        wid = lax.axis_index("s") * NC + lax.axis_index("c")
        base = wid * b_per_w
        pltpu.sync_copy(idx_hbm.at[pl.ds(pl.multiple_of(base, 8), b_per_w)],
                        idx_v)

        def fire_gathers(c, b):
            for q in range(spc):
                for (po, pw) in pieces:
                    o = pl.multiple_of(c * chunk + q * S1 + po, 8)
                    pltpu.async_copy(
                        table_hbm.at[idx_v.at[pl.ds(o, pw)]],
                        rows[b].at[q, pl.ds(po, pw)],
                        gsem[b],
                    )

        def wait_gathers(b):
            pltpu.make_async_copy(
                table_hbm.at[idx_v.at[pl.ds(0, 8)]], rows[b], gsem[b]
            ).wait()

        def fire_store(c, b):
            s0_off = pl.multiple_of((base + c * chunk) // S1, spc)
            pltpu.async_copy(rows[b], out_hbm.at[pl.ds(s0_off, spc)], ssem[b])

        def wait_store(b):
            pltpu.make_async_copy(
                rows[b], out_hbm.at[pl.ds(0, spc)], ssem[b]
            ).wait()

        fire_gathers(0, 0)

        def body(i, carry):
            g = i * 2
            for b in range(2):
                cur = g + b
                nxt = cur + 1

                @pl.when(nxt < n_chunks)
                def _():
                    @pl.when(nxt >= 2)
                    def _():
                        wait_store(b ^ 1)

                    fire_gathers(nxt, b ^ 1)

                wait_gathers(b)
                fire_store(cur, b)
            return carry

        lax.fori_loop(0, n_chunks // 2, body, 0)
        wait_store(0)
        wait_store(1)

    return gather


def kernel(inputs, table):
    S0, S1 = inputs.shape
    V, D = table.shape
    s1a = 96
    idx_a = inputs[:, :s1a].reshape(S0 * s1a).astype(jnp.int32)
    idx_b = inputs[:, s1a:].reshape(S0 * (S1 - s1a)).astype(jnp.int32)
    out_a = _make_gather(S0, s1a, V, D, spc=4)(table, idx_a)
    out_b = _make_gather(S0, S1 - s1a, V, D, spc=4)(table, idx_b)
    return jnp.concatenate([out_a, out_b], axis=1)


# speedup vs baseline: 1.1903x; 1.1903x over previous
import functools

import jax
import jax.numpy as jnp
from jax import lax
from jax.experimental import pallas as pl
from jax.experimental.pallas import tpu as pltpu
from jax.experimental.pallas import tpu_sc as plsc


def _make_gather(S0: int, S1: int, V: int, D: int, spc: int):
    info = plsc.get_sparse_core_info()
    NC, NS = info.num_cores, info.num_subcores
    NW = NC * NS
    B = S0 * S1
    chunk = spc * S1
    b_per_w = B // NW
    n_chunks = b_per_w // chunk
    assert B % (NW * chunk) == 0 and n_chunks % 2 == 0
    assert S1 % 8 == 0 and b_per_w % 8 == 0
    pieces = []
    off = 0
    while off < S1:
        w = min(128, S1 - off)
        pieces.append((off, w))
        off += w

    mesh = plsc.VectorSubcoreMesh(core_axis_name="c", subcore_axis_name="s")

    @functools.partial(
        pl.kernel,
        mesh=mesh,
        compiler_params=pltpu.CompilerParams(use_tc_tiling_on_sc=False),
        out_type=jax.ShapeDtypeStruct((S0, S1, D), jnp.float32),
        scratch_types=[
            pltpu.VMEM((b_per_w,), jnp.int32),
            pltpu.VMEM((spc, S1, D), jnp.float32),
            pltpu.VMEM((spc, S1, D), jnp.float32),
            pltpu.SemaphoreType.DMA,
            pltpu.SemaphoreType.DMA,
            pltpu.SemaphoreType.DMA,
            pltpu.SemaphoreType.DMA,
        ],
    )
    def gather(table_hbm, idx_hbm, out_hbm, idx_v, rows0, rows1,
               gsem0, gsem1, ssem0, ssem1):
        rows = (rows0, rows1)
        gsem = (gsem0, gsem1)
        ssem = (ssem0, ssem1)

        wid = lax.axis_index("s") * NC + lax.axis_index("c")
        base = wid * b_per_w
        pltpu.sync_copy(idx_hbm.at[pl.ds(pl.multiple_of(base, 8), b_per_w)],
                        idx_v)

        def fire_gathers(c, b):
            for q in range(spc):
                for (po, pw) in pieces:
                    o = pl.multiple_of(c * chunk + q * S1 + po, 8)
                    pltpu.async_copy(
                        table_hbm.at[idx_v.at[pl.ds(o, pw)]],
                        rows[b].at[q, pl.ds(po, pw)],
                        gsem[b],
                    )

        def wait_gathers(b):
            pltpu.make_async_copy(
                table_hbm.at[idx_v.at[pl.ds(0, 8)]], rows[b], gsem[b]
            ).wait()

        def fire_store(c, b):
            s0_off = pl.multiple_of((base + c * chunk) // S1, spc)
            pltpu.async_copy(rows[b], out_hbm.at[pl.ds(s0_off, spc)], ssem[b])

        def wait_store(b):
            pltpu.make_async_copy(
                rows[b], out_hbm.at[pl.ds(0, spc)], ssem[b]
            ).wait()

        fire_gathers(0, 0)

        def body(i, carry):
            g = i * 2
            for b in range(2):
                cur = g + b
                nxt = cur + 1

                @pl.when(nxt < n_chunks)
                def _():
                    @pl.when(nxt >= 2)
                    def _():
                        wait_store(b ^ 1)

                    fire_gathers(nxt, b ^ 1)

                wait_gathers(b)
                fire_store(cur, b)
            return carry

        lax.fori_loop(0, n_chunks // 2, body, 0)
        wait_store(0)
        wait_store(1)

    return gather


def kernel(inputs, table):
    S0, S1 = inputs.shape
    V, D = table.shape
    idx = inputs.reshape(S0 * S1).astype(jnp.int32)
    return _make_gather(S0, S1, V, D, spc=4)(table, idx)
